# Initial kernel scaffold; baseline (speedup 1.0000x reference)
import jax
import jax.numpy as jnp
from jax.experimental import pallas as pl


def kernel(user_ids, movie_ids, user_table, movie_table, W1, b1, g1, be1, W2, b2, g2, be2, W3, b3, g3, be3, W4, b4, global_bias):
    raise NotImplementedError("write your pallas kernel here")


# speedup vs baseline: 4.2457x; 4.2457x over previous
import functools

import jax
import jax.numpy as jnp
from jax import lax
from jax.experimental import pallas as pl
from jax.experimental.pallas import tpu as pltpu
from jax.experimental.pallas import tpu_sc as plsc

BATCH = 16384
DIM = 128
_EPS = 1e-5

_NC = 2
_NS = 16
_NW = _NC * _NS
_BPW = BATCH // _NW
_CH = 128
_NCHUNK = _BPW // _CH


def _gather_body(uids, mids, utab, mtab, uout, mout, idx_v, rows_v, sem):
    wid = lax.axis_index("s") * _NC + lax.axis_index("c")
    base = wid * _BPW
    for ids_hbm, tab, out in ((uids, utab, uout), (mids, mtab, mout)):
        pltpu.sync_copy(ids_hbm.at[wid], idx_v)
        copies = [
            pltpu.async_copy(tab.at[idx_v.at[j]],
                             rows_v.at[pl.ds(j * _CH, _CH)], sem)
            for j in range(_NCHUNK)
        ]
        for c in copies:
            c.wait()
        pltpu.sync_copy(rows_v, out.at[pl.ds(base, _BPW)])


@functools.cache
def _gather_embeddings():
    return functools.partial(
        pl.kernel,
        mesh=plsc.VectorSubcoreMesh(core_axis_name="c", subcore_axis_name="s"),
        out_type=[jax.ShapeDtypeStruct((BATCH, DIM), jnp.float32),
                  jax.ShapeDtypeStruct((BATCH, DIM), jnp.float32)],
        scratch_types=[
            pltpu.VMEM((_NCHUNK, _CH), jnp.int32),
            pltpu.VMEM((_BPW, DIM), jnp.float32),
            pltpu.SemaphoreType.DMA,
        ],
    )(_gather_body)


def _bn_relu(x, g, be):
    mu = jnp.mean(x, axis=0, keepdims=True)
    xc = x - mu
    var = jnp.mean(xc * xc, axis=0, keepdims=True)
    return jnp.maximum(xc * (g * lax.rsqrt(var + _EPS)) + be, 0.0)


def _mlp_body(u, m, w1a, w1b, b1, g1, be1, w2, b2, g2, be2,
              w3, b3, g3, be3, w4, b4, out):
    f32 = jnp.float32
    x = (jnp.dot(u[...], w1a[...], preferred_element_type=f32)
         + jnp.dot(m[...], w1b[...], preferred_element_type=f32) + b1[...])
    x = _bn_relu(x, g1[...], be1[...])
    x = _bn_relu(jnp.dot(x, w2[...], preferred_element_type=f32) + b2[...],
                 g2[...], be2[...])
    x = _bn_relu(jnp.dot(x, w3[...], preferred_element_type=f32) + b3[...],
                 g3[...], be3[...])
    out[...] = jnp.dot(x, w4[...], preferred_element_type=f32) + b4[...]


def kernel(user_ids, movie_ids, user_table, movie_table,
           W1, b1, g1, be1, W2, b2, g2, be2, W3, b3, g3, be3,
           W4, b4, global_bias):
    uids = user_ids.astype(jnp.int32).reshape(_NW, _NCHUNK, _CH)
    mids = movie_ids.astype(jnp.int32).reshape(_NW, _NCHUNK, _CH)
    u_emb, m_emb = _gather_embeddings()(uids, mids, user_table, movie_table)

    r = lambda v: v.reshape(1, -1)
    bias4 = (b4 + global_bias).reshape(1, 1)
    out = pl.pallas_call(
        _mlp_body,
        out_shape=jax.ShapeDtypeStruct((BATCH, 1), jnp.float32),
    )(u_emb, m_emb, W1[:DIM], W1[DIM:], r(b1), r(g1), r(be1),
      W2, r(b2), r(g2), r(be2), W3, r(b3), r(g3), r(be3), W4, bias4)
    return out[:, 0]


# speedup vs baseline: 4.3799x; 1.0316x over previous
import functools

import jax
import jax.numpy as jnp
from jax import lax
from jax.experimental import pallas as pl
from jax.experimental.pallas import tpu as pltpu
from jax.experimental.pallas import tpu_sc as plsc

BATCH = 16384
DIM = 128
H1 = 256
_EPS = 1e-5

_NC = 2
_NS = 16
_NW = _NC * _NS
_BPW = BATCH // _NW
_CH = 128
_NCHUNK = _BPW // _CH
_UNIT = 2 * _CH


def _gather_body(uids, mids, utab, mtab, uout, mout,
                 idxu, idxm, buf0, buf1, gsem, wsem):
    wid = lax.axis_index("s") * _NC + lax.axis_index("c")
    base = wid * _BPW
    pltpu.sync_copy(uids.at[wid], idxu)
    pltpu.sync_copy(mids.at[wid], idxm)
    units = [(utab, idxu, uout, 0), (utab, idxu, uout, 1),
             (mtab, idxm, mout, 0), (mtab, idxm, mout, 1)]
    bufs = (buf0, buf1)
    writes = [None, None]
    for i, (tab, idx, out, u) in enumerate(units):
        b = i % 2
        if writes[b] is not None:
            writes[b].wait()
        g0 = pltpu.async_copy(tab.at[idx.at[2 * u]],
                              bufs[b].at[pl.ds(0, _CH)], gsem)
        g1 = pltpu.async_copy(tab.at[idx.at[2 * u + 1]],
                              bufs[b].at[pl.ds(_CH, _CH)], gsem)
        g0.wait()
        g1.wait()
        writes[b] = pltpu.async_copy(
            bufs[b], out.at[pl.ds(base + u * _UNIT, _UNIT)], wsem)
    writes[0].wait()
    writes[1].wait()


@functools.cache
def _gather_embeddings():
    return functools.partial(
        pl.kernel,
        mesh=plsc.VectorSubcoreMesh(core_axis_name="c", subcore_axis_name="s"),
        out_type=[jax.ShapeDtypeStruct((BATCH, DIM), jnp.float32),
                  jax.ShapeDtypeStruct((BATCH, DIM), jnp.float32)],
        scratch_types=[
            pltpu.VMEM((_NCHUNK, _CH), jnp.int32),
            pltpu.VMEM((_NCHUNK, _CH), jnp.int32),
            pltpu.VMEM((_UNIT, DIM), jnp.float32),
            pltpu.VMEM((_UNIT, DIM), jnp.float32),
            pltpu.SemaphoreType.DMA,
            pltpu.SemaphoreType.DMA,
        ],
    )(_gather_body)


_GB = 2048
_NG = BATCH // _GB


def _bn_relu(x, g, be):
    mu = jnp.mean(x, axis=0, keepdims=True)
    xc = x - mu
    var = jnp.mean(xc * xc, axis=0, keepdims=True)
    return jnp.maximum(xc * (g * lax.rsqrt(var + _EPS)) + be, 0.0)


def _mlp_body(u, m, w1a, w1b, b1, g1, be1, w2, b2, g2, be2,
              w3, b3, g3, be3, w4, b4, out, h1):
    f32 = jnp.float32
    step = pl.program_id(0)
    blk = (jnp.dot(u[...], w1a[...], preferred_element_type=f32)
           + jnp.dot(m[...], w1b[...], preferred_element_type=f32) + b1[...])
    h1[pl.ds(step * _GB, _GB), :] = blk

    @pl.when(step == _NG - 1)
    def _tail():
        x = _bn_relu(h1[...], g1[...], be1[...])
        x = _bn_relu(jnp.dot(x, w2[...], preferred_element_type=f32) + b2[...],
                     g2[...], be2[...])
        x = _bn_relu(jnp.dot(x, w3[...], preferred_element_type=f32) + b3[...],
                     g3[...], be3[...])
        out[...] = jnp.dot(x, w4[...], preferred_element_type=f32) + b4[...]


def _full(shape):
    return pl.BlockSpec(shape, lambda g: (0, 0))


def kernel(user_ids, movie_ids, user_table, movie_table,
           W1, b1, g1, be1, W2, b2, g2, be2, W3, b3, g3, be3,
           W4, b4, global_bias):
    uids = user_ids.astype(jnp.int32).reshape(_NW, _NCHUNK, _CH)
    mids = movie_ids.astype(jnp.int32).reshape(_NW, _NCHUNK, _CH)
    u_emb, m_emb = _gather_embeddings()(uids, mids, user_table, movie_table)

    r = lambda v: v.reshape(1, -1)
    bias4 = (b4 + global_bias).reshape(1, 1)
    blk = pl.BlockSpec((_GB, DIM), lambda g: (g, 0))
    out = pl.pallas_call(
        _mlp_body,
        grid=(_NG,),
        in_specs=[blk, blk,
                  _full((DIM, H1)), _full((DIM, H1)), _full((1, H1)),
                  _full((1, H1)), _full((1, H1)),
                  _full((H1, 128)), _full((1, 128)), _full((1, 128)),
                  _full((1, 128)),
                  _full((128, 64)), _full((1, 64)), _full((1, 64)),
                  _full((1, 64)),
                  _full((64, 1)), _full((1, 1))],
        out_specs=_full((BATCH, 1)),
        out_shape=jax.ShapeDtypeStruct((BATCH, 1), jnp.float32),
        scratch_shapes=[pltpu.VMEM((BATCH, H1), jnp.float32)],
    )(u_emb, m_emb, W1[:DIM], W1[DIM:], r(b1), r(g1), r(be1),
      W2, r(b2), r(g2), r(be2), W3, r(b3), r(g3), r(be3), W4, bias4)
    return out[:, 0]


# speedup vs baseline: 5.1451x; 1.1747x over previous
import functools

import jax
import jax.numpy as jnp
from jax import lax
from jax.experimental import pallas as pl
from jax.experimental.pallas import tpu as pltpu
from jax.experimental.pallas import tpu_sc as plsc

BATCH = 16384
DIM = 128
H1 = 256
_EPS = 1e-5

_NC = 2
_NS = 16
_NW = _NC * _NS
_BPW = BATCH // _NW
_CH = 128
_NCHUNK = _BPW // _CH
_UNIT = 2 * _CH


def _gather_body(uids, mids, utab, mtab, uout, mout,
                 idxu, idxm, buf0, buf1, gsem, wsem):
    wid = lax.axis_index("s") * _NC + lax.axis_index("c")
    base = wid * _BPW
    pltpu.sync_copy(uids.at[wid], idxu)
    pltpu.sync_copy(mids.at[wid], idxm)
    units = [(utab, idxu, uout, 0), (utab, idxu, uout, 1),
             (mtab, idxm, mout, 0), (mtab, idxm, mout, 1)]
    bufs = (buf0, buf1)
    writes = [None, None]
    for i, (tab, idx, out, u) in enumerate(units):
        b = i % 2
        if writes[b] is not None:
            writes[b].wait()
        g0 = pltpu.async_copy(tab.at[idx.at[2 * u]],
                              bufs[b].at[pl.ds(0, _CH)], gsem)
        g1 = pltpu.async_copy(tab.at[idx.at[2 * u + 1]],
                              bufs[b].at[pl.ds(_CH, _CH)], gsem)
        g0.wait()
        g1.wait()
        writes[b] = pltpu.async_copy(
            bufs[b], out.at[pl.ds(base + u * _UNIT, _UNIT)], wsem)
    writes[0].wait()
    writes[1].wait()


@functools.cache
def _gather_embeddings():
    return functools.partial(
        pl.kernel,
        mesh=plsc.VectorSubcoreMesh(core_axis_name="c", subcore_axis_name="s"),
        out_type=[jax.ShapeDtypeStruct((BATCH, DIM), jnp.float32),
                  jax.ShapeDtypeStruct((BATCH, DIM), jnp.float32)],
        scratch_types=[
            pltpu.VMEM((_NCHUNK, _CH), jnp.int32),
            pltpu.VMEM((_NCHUNK, _CH), jnp.int32),
            pltpu.VMEM((_UNIT, DIM), jnp.float32),
            pltpu.VMEM((_UNIT, DIM), jnp.float32),
            pltpu.SemaphoreType.DMA,
            pltpu.SemaphoreType.DMA,
        ],
    )(_gather_body)


_GB = 2048
_NG = BATCH // _GB


def _bn_relu_stats(y, g, be, mu, var):
    scale = g * lax.rsqrt(var + _EPS)
    shift = be - mu * scale
    return jnp.maximum(y * scale + shift, 0.0)


def _mlp_body(u, m, w1, b1, g1, be1, w2, b2, g2, be2,
              w3, b3, g3, be3, w4, b4, out, h1, s1, s2):
    f32 = jnp.float32
    step = pl.program_id(0)
    w1v = w1[...]
    blk = (jnp.dot(u[...], lax.slice(w1v, (0, 0), (DIM, H1)),
                   preferred_element_type=f32)
           + jnp.dot(m[...], lax.slice(w1v, (DIM, 0), (2 * DIM, H1)),
                     preferred_element_type=f32) + b1[...])
    h1[pl.ds(step * _GB, _GB), :] = blk
    ps1 = jnp.sum(blk, axis=0, keepdims=True)
    ps2 = jnp.sum(blk * blk, axis=0, keepdims=True)

    @pl.when(step == 0)
    def _init():
        s1[...] = ps1
        s2[...] = ps2

    @pl.when(step > 0)
    def _acc():
        s1[...] += ps1
        s2[...] += ps2

    @pl.when(step == _NG - 1)
    def _tail():
        inv_n = 1.0 / BATCH
        mu1 = s1[...] * inv_n
        var1 = s2[...] * inv_n - mu1 * mu1
        x = _bn_relu_stats(h1[...], g1[...], be1[...], mu1, var1)

        y = jnp.dot(x, w2[...], preferred_element_type=f32) + b2[...]
        mu2 = jnp.mean(y, axis=0, keepdims=True)
        var2 = jnp.mean(y * y, axis=0, keepdims=True) - mu2 * mu2
        y = _bn_relu_stats(y, g2[...], be2[...], mu2, var2)

        z = jnp.dot(y, w3[...], preferred_element_type=f32) + b3[...]
        mu3 = jnp.mean(z, axis=0, keepdims=True)
        var3 = jnp.mean(z * z, axis=0, keepdims=True) - mu3 * mu3
        z = _bn_relu_stats(z, g3[...], be3[...], mu3, var3)

        res = jnp.dot(z, w4[...], preferred_element_type=f32) + b4[...]
        out[...] = res[:, 0]


def _full(shape):
    return pl.BlockSpec(shape, lambda g: tuple(0 for _ in shape))


def kernel(user_ids, movie_ids, user_table, movie_table,
           W1, b1, g1, be1, W2, b2, g2, be2, W3, b3, g3, be3,
           W4, b4, global_bias):
    uids = user_ids.astype(jnp.int32).reshape(_NW, _NCHUNK, _CH)
    mids = movie_ids.astype(jnp.int32).reshape(_NW, _NCHUNK, _CH)
    u_emb, m_emb = _gather_embeddings()(uids, mids, user_table, movie_table)

    r = lambda v: v.reshape(1, -1)
    bias4 = (b4 + global_bias).reshape(1, 1)
    blk = pl.BlockSpec((_GB, DIM), lambda g: (g, 0))
    out = pl.pallas_call(
        _mlp_body,
        grid=(_NG,),
        in_specs=[blk, blk,
                  _full((2 * DIM, H1)), _full((1, H1)),
                  _full((1, H1)), _full((1, H1)),
                  _full((H1, 128)), _full((1, 128)), _full((1, 128)),
                  _full((1, 128)),
                  _full((128, 64)), _full((1, 64)), _full((1, 64)),
                  _full((1, 64)),
                  _full((64, 1)), _full((1, 1))],
        out_specs=_full((BATCH,)),
        out_shape=jax.ShapeDtypeStruct((BATCH,), jnp.float32),
        scratch_shapes=[pltpu.VMEM((BATCH, H1), jnp.float32),
                        pltpu.VMEM((1, H1), jnp.float32),
                        pltpu.VMEM((1, H1), jnp.float32)],
    )(u_emb, m_emb, W1, r(b1), r(g1), r(be1),
      W2, r(b2), r(g2), r(be2), W3, r(b3), r(g3), r(be3), W4, bias4)
    return out


# speedup vs baseline: 5.1998x; 1.0106x over previous
import functools

import jax
import jax.numpy as jnp
from jax import lax
from jax.experimental import pallas as pl
from jax.experimental.pallas import tpu as pltpu
from jax.experimental.pallas import tpu_sc as plsc

BATCH = 16384
DIM = 128
H1 = 256
_EPS = 1e-5

_NC = 2
_NS = 16
_NW = _NC * _NS
_BPW = BATCH // _NW
_CH = 128
_NCHUNK = _BPW // _CH
_UNIT = 2 * _CH


def _gather_body(uids, mids, utab, mtab, uout, mout,
                 idxu, idxm, buf0, buf1, gsem, wsem):
    wid = lax.axis_index("s") * _NC + lax.axis_index("c")
    base = wid * _BPW
    pltpu.sync_copy(uids.at[wid], idxu)
    pltpu.sync_copy(mids.at[wid], idxm)
    units = [(utab, idxu, uout, 0), (utab, idxu, uout, 1),
             (mtab, idxm, mout, 0), (mtab, idxm, mout, 1)]
    bufs = (buf0, buf1)
    writes = [None, None]
    for i, (tab, idx, out, u) in enumerate(units):
        b = i % 2
        if writes[b] is not None:
            writes[b].wait()
        g0 = pltpu.async_copy(tab.at[idx.at[2 * u]],
                              bufs[b].at[pl.ds(0, _CH)], gsem)
        g1 = pltpu.async_copy(tab.at[idx.at[2 * u + 1]],
                              bufs[b].at[pl.ds(_CH, _CH)], gsem)
        g0.wait()
        g1.wait()
        writes[b] = pltpu.async_copy(
            bufs[b], out.at[pl.ds(base + u * _UNIT, _UNIT)], wsem)
    writes[0].wait()
    writes[1].wait()


@functools.cache
def _gather_embeddings():
    return functools.partial(
        pl.kernel,
        mesh=plsc.VectorSubcoreMesh(core_axis_name="c", subcore_axis_name="s"),
        out_type=[jax.ShapeDtypeStruct((BATCH, DIM), jnp.float32),
                  jax.ShapeDtypeStruct((BATCH, DIM), jnp.float32)],
        scratch_types=[
            pltpu.VMEM((_NCHUNK, _CH), jnp.int32),
            pltpu.VMEM((_NCHUNK, _CH), jnp.int32),
            pltpu.VMEM((_UNIT, DIM), jnp.float32),
            pltpu.VMEM((_UNIT, DIM), jnp.float32),
            pltpu.SemaphoreType.DMA,
            pltpu.SemaphoreType.DMA,
        ],
    )(_gather_body)


_GB = 2048
_NG = BATCH // (2 * _GB)
_HALF = BATCH // 2


def _bn_relu_stats(y, g, be, mu, var):
    scale = g * lax.rsqrt(var + _EPS)
    shift = be - mu * scale
    return jnp.maximum(y * scale + shift, 0.0)


def _mlp_body(ulo, uhi, mlo, mhi, w1, b1, g1, be1, w2, b2, g2, be2,
              w3, b3, g3, be3, w4, b4, out, h1, s1, s2):
    f32 = jnp.float32
    bf = jnp.bfloat16
    step = pl.program_id(0)
    w1v = w1[...].astype(bf)
    w1a = lax.slice(w1v, (0, 0), (DIM, H1))
    w1b = lax.slice(w1v, (DIM, 0), (2 * DIM, H1))
    blk_lo = (jnp.dot(ulo[...].astype(bf), w1a, preferred_element_type=f32)
              + jnp.dot(mlo[...].astype(bf), w1b, preferred_element_type=f32)
              + b1[...])
    blk_hi = (jnp.dot(uhi[...].astype(bf), w1a, preferred_element_type=f32)
              + jnp.dot(mhi[...].astype(bf), w1b, preferred_element_type=f32)
              + b1[...])
    h1[pl.ds(step * _GB, _GB), :] = blk_lo
    h1[pl.ds(_HALF + step * _GB, _GB), :] = blk_hi
    ps1 = jnp.sum(blk_lo, axis=0, keepdims=True) \
        + jnp.sum(blk_hi, axis=0, keepdims=True)
    ps2 = jnp.sum(blk_lo * blk_lo, axis=0, keepdims=True) \
        + jnp.sum(blk_hi * blk_hi, axis=0, keepdims=True)

    @pl.when(step == 0)
    def _init():
        s1[...] = ps1
        s2[...] = ps2

    @pl.when(step > 0)
    def _acc():
        s1[...] += ps1
        s2[...] += ps2

    @pl.when(step == _NG - 1)
    def _tail():
        inv_n = 1.0 / BATCH
        mu1 = s1[...] * inv_n
        var1 = s2[...] * inv_n - mu1 * mu1
        x = _bn_relu_stats(h1[...], g1[...], be1[...], mu1, var1)

        y = jnp.dot(x, w2[...], preferred_element_type=f32) + b2[...]
        mu2 = jnp.mean(y, axis=0, keepdims=True)
        var2 = jnp.mean(y * y, axis=0, keepdims=True) - mu2 * mu2
        y = _bn_relu_stats(y, g2[...], be2[...], mu2, var2)

        z = jnp.dot(y, w3[...], preferred_element_type=f32) + b3[...]
        mu3 = jnp.mean(z, axis=0, keepdims=True)
        var3 = jnp.mean(z * z, axis=0, keepdims=True) - mu3 * mu3
        z = _bn_relu_stats(z, g3[...], be3[...], mu3, var3)

        res = jnp.dot(z, w4[...], preferred_element_type=f32) + b4[...]
        out[...] = res[:, 0]


def _full(shape):
    return pl.BlockSpec(shape, lambda g: tuple(0 for _ in shape))


def kernel(user_ids, movie_ids, user_table, movie_table,
           W1, b1, g1, be1, W2, b2, g2, be2, W3, b3, g3, be3,
           W4, b4, global_bias):
    uids = user_ids.astype(jnp.int32).reshape(_NW, _NCHUNK, _CH)
    mids = movie_ids.astype(jnp.int32).reshape(_NW, _NCHUNK, _CH)
    u_emb, m_emb = _gather_embeddings()(uids, mids, user_table, movie_table)

    r = lambda v: v.reshape(1, -1)
    bias4 = (b4 + global_bias).reshape(1, 1)
    lo = pl.BlockSpec((_GB, DIM), lambda g: (g, 0))
    hi = pl.BlockSpec((_GB, DIM), lambda g: (g + _NG, 0))
    out = pl.pallas_call(
        _mlp_body,
        grid=(_NG,),
        in_specs=[lo, hi, lo, hi,
                  _full((2 * DIM, H1)), _full((1, H1)),
                  _full((1, H1)), _full((1, H1)),
                  _full((H1, 128)), _full((1, 128)), _full((1, 128)),
                  _full((1, 128)),
                  _full((128, 64)), _full((1, 64)), _full((1, 64)),
                  _full((1, 64)),
                  _full((64, 1)), _full((1, 1))],
        out_specs=_full((BATCH,)),
        out_shape=jax.ShapeDtypeStruct((BATCH,), jnp.float32),
        scratch_shapes=[pltpu.VMEM((BATCH, H1), jnp.float32),
                        pltpu.VMEM((1, H1), jnp.float32),
                        pltpu.VMEM((1, H1), jnp.float32)],
    )(u_emb, u_emb, m_emb, m_emb, W1, r(b1), r(g1), r(be1),
      W2, r(b2), r(g2), r(be2), W3, r(b3), r(g3), r(be3), W4, bias4)
    return out


# speedup vs baseline: 5.7331x; 1.1026x over previous
import functools

import jax
import jax.numpy as jnp
from jax import lax
from jax.experimental import pallas as pl
from jax.experimental.pallas import tpu as pltpu
from jax.experimental.pallas import tpu_sc as plsc

BATCH = 16384
DIM = 128
H1 = 256
_EPS = 1e-5

_NC = 2
_NS = 16
_NW = _NC * _NS
_BPW = BATCH // _NW
_CH = 128
_NCHUNK = _BPW // _CH
_UNIT = 2 * _CH


def _gather_body(uids, mids, utab, mtab, uout, mout,
                 idxu, idxm, buf0, buf1, gsem, wsem):
    wid = lax.axis_index("s") * _NC + lax.axis_index("c")
    base = wid * _BPW
    pltpu.sync_copy(uids.at[wid], idxu)
    pltpu.sync_copy(mids.at[wid], idxm)
    units = [(utab, idxu, uout, 0), (utab, idxu, uout, 1),
             (mtab, idxm, mout, 0), (mtab, idxm, mout, 1)]
    bufs = (buf0, buf1)
    writes = [None, None]
    for i, (tab, idx, out, u) in enumerate(units):
        b = i % 2
        if writes[b] is not None:
            writes[b].wait()
        g0 = pltpu.async_copy(tab.at[idx.at[2 * u]],
                              bufs[b].at[pl.ds(0, _CH)], gsem)
        g1 = pltpu.async_copy(tab.at[idx.at[2 * u + 1]],
                              bufs[b].at[pl.ds(_CH, _CH)], gsem)
        g0.wait()
        g1.wait()
        writes[b] = pltpu.async_copy(
            bufs[b], out.at[pl.ds(base + u * _UNIT, _UNIT)], wsem)
    writes[0].wait()
    writes[1].wait()


@functools.cache
def _gather_embeddings():
    return functools.partial(
        pl.kernel,
        mesh=plsc.VectorSubcoreMesh(core_axis_name="c", subcore_axis_name="s"),
        out_type=[jax.ShapeDtypeStruct((BATCH, DIM), jnp.float32),
                  jax.ShapeDtypeStruct((BATCH, DIM), jnp.float32)],
        scratch_types=[
            pltpu.VMEM((_NCHUNK, _CH), jnp.int32),
            pltpu.VMEM((_NCHUNK, _CH), jnp.int32),
            pltpu.VMEM((_UNIT, DIM), jnp.float32),
            pltpu.VMEM((_UNIT, DIM), jnp.float32),
            pltpu.SemaphoreType.DMA,
            pltpu.SemaphoreType.DMA,
        ],
    )(_gather_body)


_GB = 2048
_NG = BATCH // (2 * _GB)
_HALF = BATCH // 2


def _bn_relu_stats(y, g, be, mu, var):
    scale = g * lax.rsqrt(var + _EPS)
    shift = be - mu * scale
    return jnp.maximum(y * scale + shift, 0.0)


def _mlp_body(ulo, uhi, mlo, mhi, w1, b1, g1, be1, w2, b2, g2, be2,
              w3, b3, g3, be3, w4t, b4, out, h1, s1, s2):
    f32 = jnp.float32
    bf = jnp.bfloat16
    step = pl.program_id(0)
    w1v = w1[...].astype(bf)
    w1a = lax.slice(w1v, (0, 0), (DIM, H1))
    w1b = lax.slice(w1v, (DIM, 0), (2 * DIM, H1))
    blk_lo = (jnp.dot(ulo[...].astype(bf), w1a, preferred_element_type=f32)
              + jnp.dot(mlo[...].astype(bf), w1b, preferred_element_type=f32)
              + b1[...])
    blk_hi = (jnp.dot(uhi[...].astype(bf), w1a, preferred_element_type=f32)
              + jnp.dot(mhi[...].astype(bf), w1b, preferred_element_type=f32)
              + b1[...])
    h1[pl.ds(step * _GB, _GB), :] = blk_lo
    h1[pl.ds(_HALF + step * _GB, _GB), :] = blk_hi
    ps1 = jnp.sum(blk_lo, axis=0, keepdims=True) \
        + jnp.sum(blk_hi, axis=0, keepdims=True)
    ps2 = jnp.sum(blk_lo * blk_lo, axis=0, keepdims=True) \
        + jnp.sum(blk_hi * blk_hi, axis=0, keepdims=True)

    @pl.when(step == 0)
    def _init():
        s1[...] = ps1
        s2[...] = ps2

    @pl.when(step > 0)
    def _acc():
        s1[...] += ps1
        s2[...] += ps2

    @pl.when(step == _NG - 1)
    def _tail():
        inv_n = 1.0 / BATCH
        mu1 = s1[...] * inv_n
        var1 = s2[...] * inv_n - mu1 * mu1
        x = _bn_relu_stats(h1[...], g1[...], be1[...], mu1, var1)

        y = jnp.dot(x.astype(bf), w2[...].astype(bf),
                    preferred_element_type=f32) + b2[...]
        mu2 = jnp.mean(y, axis=0, keepdims=True)
        var2 = jnp.mean(y * y, axis=0, keepdims=True) - mu2 * mu2
        y = _bn_relu_stats(y, g2[...], be2[...], mu2, var2)

        z = jnp.dot(y.astype(bf), w3[...].astype(bf),
                    preferred_element_type=f32) + b3[...]
        mu3 = jnp.mean(z, axis=0, keepdims=True)
        var3 = jnp.mean(z * z, axis=0, keepdims=True) - mu3 * mu3
        z = _bn_relu_stats(z, g3[...], be3[...], mu3, var3)

        zt = jnp.transpose(z)
        res = jnp.dot(w4t[...], zt, preferred_element_type=f32) + b4[...]
        out[...] = jnp.reshape(res, (BATCH,))


def _full(shape):
    return pl.BlockSpec(shape, lambda g: tuple(0 for _ in shape))


def kernel(user_ids, movie_ids, user_table, movie_table,
           W1, b1, g1, be1, W2, b2, g2, be2, W3, b3, g3, be3,
           W4, b4, global_bias):
    uids = user_ids.astype(jnp.int32).reshape(_NW, _NCHUNK, _CH)
    mids = movie_ids.astype(jnp.int32).reshape(_NW, _NCHUNK, _CH)
    u_emb, m_emb = _gather_embeddings()(uids, mids, user_table, movie_table)

    r = lambda v: v.reshape(1, -1)
    bias4 = (b4 + global_bias).reshape(1, 1)
    lo = pl.BlockSpec((_GB, DIM), lambda g: (g, 0))
    hi = pl.BlockSpec((_GB, DIM), lambda g: (g + _NG, 0))
    out = pl.pallas_call(
        _mlp_body,
        grid=(_NG,),
        in_specs=[lo, hi, lo, hi,
                  _full((2 * DIM, H1)), _full((1, H1)),
                  _full((1, H1)), _full((1, H1)),
                  _full((H1, 128)), _full((1, 128)), _full((1, 128)),
                  _full((1, 128)),
                  _full((128, 64)), _full((1, 64)), _full((1, 64)),
                  _full((1, 64)),
                  _full((1, 64)), _full((1, 1))],
        out_specs=_full((BATCH,)),
        out_shape=jax.ShapeDtypeStruct((BATCH,), jnp.float32),
        scratch_shapes=[pltpu.VMEM((BATCH, H1), jnp.float32),
                        pltpu.VMEM((1, H1), jnp.float32),
                        pltpu.VMEM((1, H1), jnp.float32)],
    )(u_emb, u_emb, m_emb, m_emb, W1, r(b1), r(g1), r(be1),
      W2, r(b2), r(g2), r(be2), W3, r(b3), r(g3), r(be3),
      W4.reshape(1, -1), bias4)
    return out


# speedup vs baseline: 5.7361x; 1.0005x over previous
import functools

import jax
import jax.numpy as jnp
from jax import lax
from jax.experimental import pallas as pl
from jax.experimental.pallas import tpu as pltpu
from jax.experimental.pallas import tpu_sc as plsc

BATCH = 16384
DIM = 128
H1 = 256
_EPS = 1e-5

_NC = 2
_NS = 16
_NW = _NC * _NS
_BPW = BATCH // _NW
_CH = 128
_NCHUNK = _BPW // _CH
_UNIT = 2 * _CH


def _gather_body(uids, mids, utab, mtab, uout, mout,
                 idxu, idxm, buf0, buf1, gsem, wsem):
    wid = lax.axis_index("s") * _NC + lax.axis_index("c")
    base = wid * _BPW
    pltpu.sync_copy(uids.at[wid], idxu)
    pltpu.sync_copy(mids.at[wid], idxm)
    units = [(utab, idxu, uout, 0), (utab, idxu, uout, 1),
             (mtab, idxm, mout, 0), (mtab, idxm, mout, 1)]
    bufs = (buf0, buf1)
    writes = [None, None]
    for i, (tab, idx, out, u) in enumerate(units):
        b = i % 2
        if writes[b] is not None:
            writes[b].wait()
        g0 = pltpu.async_copy(tab.at[idx.at[2 * u]],
                              bufs[b].at[pl.ds(0, _CH)], gsem)
        g1 = pltpu.async_copy(tab.at[idx.at[2 * u + 1]],
                              bufs[b].at[pl.ds(_CH, _CH)], gsem)
        g0.wait()
        g1.wait()
        writes[b] = pltpu.async_copy(
            bufs[b], out.at[pl.ds(base + u * _UNIT, _UNIT)], wsem)
    writes[0].wait()
    writes[1].wait()


@functools.cache
def _gather_embeddings():
    return functools.partial(
        pl.kernel,
        mesh=plsc.VectorSubcoreMesh(core_axis_name="c", subcore_axis_name="s"),
        out_type=[jax.ShapeDtypeStruct((BATCH, DIM), jnp.float32),
                  jax.ShapeDtypeStruct((BATCH, DIM), jnp.float32)],
        scratch_types=[
            pltpu.VMEM((_NCHUNK, _CH), jnp.int32),
            pltpu.VMEM((_NCHUNK, _CH), jnp.int32),
            pltpu.VMEM((_UNIT, DIM), jnp.float32),
            pltpu.VMEM((_UNIT, DIM), jnp.float32),
            pltpu.SemaphoreType.DMA,
            pltpu.SemaphoreType.DMA,
        ],
    )(_gather_body)


_NV = 4
_NG = 4
_GB = BATCH // (_NV * _NG)
_QTR = BATCH // _NV


def _bn_relu_stats(y, g, be, mu, var):
    scale = g * lax.rsqrt(var + _EPS)
    shift = be - mu * scale
    return jnp.maximum(y * scale + shift, 0.0)


def _mlp_body(u0, u1, u2, u3, m0, m1, m2, m3, w1, b1, g1, be1,
              w2, b2, g2, be2, w3, b3, g3, be3, w4t, b4, out, h1, s1, s2):
    f32 = jnp.float32
    bf = jnp.bfloat16
    step = pl.program_id(0)
    w1v = w1[...].astype(bf)
    w1a = lax.slice(w1v, (0, 0), (DIM, H1))
    w1b = lax.slice(w1v, (DIM, 0), (2 * DIM, H1))
    ps1 = jnp.zeros((1, H1), f32)
    ps2 = jnp.zeros((1, H1), f32)
    for q, (uq, mq) in enumerate(((u0, m0), (u1, m1), (u2, m2), (u3, m3))):
        blk = (jnp.dot(uq[...].astype(bf), w1a, preferred_element_type=f32)
               + jnp.dot(mq[...].astype(bf), w1b, preferred_element_type=f32)
               + b1[...])
        h1[pl.ds(q * _QTR + step * _GB, _GB), :] = blk
        ps1 = ps1 + jnp.sum(blk, axis=0, keepdims=True)
        ps2 = ps2 + jnp.sum(blk * blk, axis=0, keepdims=True)

    @pl.when(step == 0)
    def _init():
        s1[...] = ps1
        s2[...] = ps2

    @pl.when(step > 0)
    def _acc():
        s1[...] += ps1
        s2[...] += ps2

    @pl.when(step == _NG - 1)
    def _tail():
        inv_n = 1.0 / BATCH
        mu1 = s1[...] * inv_n
        var1 = s2[...] * inv_n - mu1 * mu1
        x = _bn_relu_stats(h1[...], g1[...], be1[...], mu1, var1)

        y = jnp.dot(x.astype(bf), w2[...].astype(bf),
                    preferred_element_type=f32) + b2[...]
        mu2 = jnp.mean(y, axis=0, keepdims=True)
        var2 = jnp.mean(y * y, axis=0, keepdims=True) - mu2 * mu2
        y = _bn_relu_stats(y, g2[...], be2[...], mu2, var2)

        z = jnp.dot(y.astype(bf), w3[...].astype(bf),
                    preferred_element_type=f32) + b3[...]
        mu3 = jnp.mean(z, axis=0, keepdims=True)
        var3 = jnp.mean(z * z, axis=0, keepdims=True) - mu3 * mu3
        z = _bn_relu_stats(z, g3[...], be3[...], mu3, var3)

        zt = jnp.transpose(z)
        res = jnp.dot(w4t[...], zt, preferred_element_type=f32) + b4[...]
        out[...] = jnp.reshape(res, (BATCH,))


def _full(shape):
    return pl.BlockSpec(shape, lambda g: tuple(0 for _ in shape))


def kernel(user_ids, movie_ids, user_table, movie_table,
           W1, b1, g1, be1, W2, b2, g2, be2, W3, b3, g3, be3,
           W4, b4, global_bias):
    uids = user_ids.astype(jnp.int32).reshape(_NW, _NCHUNK, _CH)
    mids = movie_ids.astype(jnp.int32).reshape(_NW, _NCHUNK, _CH)
    u_emb, m_emb = _gather_embeddings()(uids, mids, user_table, movie_table)

    r = lambda v: v.reshape(1, -1)
    bias4 = (b4 + global_bias).reshape(1, 1)
    qspecs = [pl.BlockSpec((_GB, DIM), lambda g, k=k: (g + k * _NG, 0))
              for k in range(_NV)]
    out = pl.pallas_call(
        _mlp_body,
        grid=(_NG,),
        in_specs=[*qspecs, *qspecs,
                  _full((2 * DIM, H1)), _full((1, H1)),
                  _full((1, H1)), _full((1, H1)),
                  _full((H1, 128)), _full((1, 128)), _full((1, 128)),
                  _full((1, 128)),
                  _full((128, 64)), _full((1, 64)), _full((1, 64)),
                  _full((1, 64)),
                  _full((1, 64)), _full((1, 1))],
        out_specs=_full((BATCH,)),
        out_shape=jax.ShapeDtypeStruct((BATCH,), jnp.float32),
        scratch_shapes=[pltpu.VMEM((BATCH, H1), jnp.float32),
                        pltpu.VMEM((1, H1), jnp.float32),
                        pltpu.VMEM((1, H1), jnp.float32)],
    )(u_emb, u_emb, u_emb, u_emb, m_emb, m_emb, m_emb, m_emb,
      W1, r(b1), r(g1), r(be1),
      W2, r(b2), r(g2), r(be2), W3, r(b3), r(g3), r(be3),
      W4.reshape(1, -1), bias4)
    return out
